# baseline (speedup 1.0000x reference)
import functools

import numpy as np
import jax
import jax.numpy as jnp
from jax import lax
from jax.experimental import pallas as pl
from jax.experimental.pallas import tpu as pltpu
from jax.experimental.pallas import tpu_sc as plsc

_BM = 512
_BN = 512

_NW = 32
_SUB = 128
_NSUB = 49
_CH = _NSUB * _SUB


def _mlp_body(x_ref, wn1, bn1, wn2, bn2, wr1, br1, wr2, br2, ws1, bs1, ws2,
              bs2, out_ref):
    x = x_ref[0]
    x6 = x[0:8].astype(jnp.bfloat16)
    xall = x.astype(jnp.bfloat16)

    def layer(w, b, xin):
        h = lax.dot_general(w[...], xin, (((1,), (0,)), ((), ())),
                            preferred_element_type=jnp.float32)
        return jnp.maximum(h + b[...], 0.0)

    hn = layer(wn1, bn1, x6)
    hr = layer(wr1, br1, x6)
    hs = layer(ws1, bs1, xall)
    fn = layer(wn2, bn2, hn.astype(jnp.bfloat16))
    fr = layer(wr2, br2, hr.astype(jnp.bfloat16))
    fs = layer(ws2, bs2, hs.astype(jnp.bfloat16))
    g = fn + fr + fs
    out_ref[0] = g.T


def _merge_body(feat_ref, c_ref, idx_ref, out_ref):
    c = c_ref[0]
    ct = c.T
    mask = idx_ref[0] >= 0
    out_ref[0] = feat_ref[0] + jnp.where(mask, ct, 0.0)


def _sc_gather_body(tot, g_hbm, idxg_hbm, out_hbm, idx_v, buf_v, sem):
    wid = lax.axis_index("s") * 2 + lax.axis_index("c")
    base = jnp.minimum(wid * _CH, tot - _CH)
    pltpu.sync_copy(idxg_hbm.at[wid], idx_v)

    def step(j, carry):
        pltpu.async_copy(g_hbm.at[idx_v.at[j]], buf_v, sem).wait()
        pltpu.sync_copy(buf_v, out_hbm.at[pl.ds(base + j * _SUB, _SUB)])
        return carry

    lax.fori_loop(0, _NSUB, step, 0)


def kernel(feat, mv_feat, pc2pc_idx, Wn1, bn1, Wn2, bn2, Wr1, br1, Wr2, br2,
           Ws1, bs1, Ws2, bs2):
    B, C, N = feat.shape
    M = mv_feat.shape[2]
    Cin = mv_feat.shape[1]

    wn1p = jnp.pad(Wn1, ((0, 0), (0, 5))).astype(jnp.bfloat16)
    wr1p = jnp.pad(Wr1, ((0, 0), (3, 2))).astype(jnp.bfloat16)
    ws1p = jnp.pad(Ws1, ((0, 0), (6, 0))).astype(jnp.bfloat16)
    wn2b = Wn2.astype(jnp.bfloat16)
    wr2b = Wr2.astype(jnp.bfloat16)
    ws2b = Ws2.astype(jnp.bfloat16)
    b2d = lambda b: b.reshape(C, 1)

    nmb = pl.cdiv(M, _BM)
    wspec = lambda shape: pl.BlockSpec(shape, lambda b, i: (0, 0))
    mlp_call = pl.pallas_call(
        _mlp_body,
        grid=(B, nmb),
        in_specs=[
            pl.BlockSpec((1, Cin, _BM), lambda b, i: (b, 0, i)),
            wspec((C, 8)), wspec((C, 1)), wspec((C, C)), wspec((C, 1)),
            wspec((C, 8)), wspec((C, 1)), wspec((C, C)), wspec((C, 1)),
            wspec((C, Cin)), wspec((C, 1)), wspec((C, C)), wspec((C, 1)),
        ],
        out_specs=pl.BlockSpec((1, _BM, C), lambda b, i: (b, i, 0)),
        out_shape=jax.ShapeDtypeStruct((B, M, C), jnp.float32),
        compiler_params=pltpu.CompilerParams(
            dimension_semantics=("parallel", "parallel")),
    )
    G = mlp_call(mv_feat, wn1p, b2d(bn1), wn2b, b2d(bn2), wr1p, b2d(br1),
                 wr2b, b2d(br2), ws1p, b2d(bs1), ws2b, b2d(bs2))
    Gf = G.reshape(B * M, C)

    idx = pc2pc_idx.reshape(B, N).astype(jnp.int32)
    idx_safe = jnp.where(idx >= 0, idx, 0) + (
        jnp.arange(B, dtype=jnp.int32) * M)[:, None]
    flat = idx_safe.reshape(B * N)
    tot = B * N
    bases = np.minimum(np.arange(_NW) * _CH, tot - _CH)
    offs = bases[:, None] + np.arange(_CH)[None, :]
    idxg = jnp.take(flat, jnp.asarray(offs, dtype=jnp.int32),
                    axis=0).reshape(_NW, _NSUB, _SUB)

    sc_gather = pl.kernel(
        functools.partial(_sc_gather_body, tot),
        out_type=jax.ShapeDtypeStruct((tot, C), jnp.float32),
        mesh=plsc.VectorSubcoreMesh(core_axis_name="c", subcore_axis_name="s"),
        scratch_types=[
            pltpu.VMEM((_NSUB, _SUB), jnp.int32),
            pltpu.VMEM((_SUB, C), jnp.float32),
            pltpu.SemaphoreType.DMA,
        ],
    )
    contrib = sc_gather(Gf, idxg)
    contribT = contrib.reshape(B, N, C)
    idx3 = idx.reshape(B, 1, N)

    nnb = pl.cdiv(N, _BN)
    merge_call = pl.pallas_call(
        _merge_body,
        grid=(B, nnb),
        in_specs=[
            pl.BlockSpec((1, C, _BN), lambda b, i: (b, 0, i)),
            pl.BlockSpec((1, _BN, C), lambda b, i: (b, i, 0)),
            pl.BlockSpec((1, 1, _BN), lambda b, i: (b, 0, i)),
        ],
        out_specs=pl.BlockSpec((1, C, _BN), lambda b, i: (b, 0, i)),
        out_shape=jax.ShapeDtypeStruct((B, C, N), jnp.float32),
        compiler_params=pltpu.CompilerParams(
            dimension_semantics=("parallel", "parallel")),
    )
    return merge_call(feat, contribT, idx3)
